# baseline (speedup 1.0000x reference)
import functools

import jax
import jax.numpy as jnp
from jax import lax
from jax.experimental import pallas as pl
from jax.experimental.pallas import tpu as pltpu
from jax.experimental.pallas import tpu_sc as plsc

NUM_EMB = 1000000
DIM = 64
N_TOK = 4096
SEQ = 50
NC = 2
NS = 16
NW = NC * NS
L = 16

TILE_C = 128
NTC = NUM_EMB // TILE_C
TAIL = NUM_EMB - NTC * TILE_C
K1_ITERS = NTC // NW + 1
BLK = TILE_C * DIM

ROWS_PER_W = N_TOK // NW
NBUF = 8
PF = NBUF - 1


def _make_k1():
    mesh = plsc.VectorSubcoreMesh(core_axis_name="c", subcore_axis_name="s")

    @functools.partial(
        pl.kernel,
        out_type=jax.ShapeDtypeStruct((NUM_EMB * DIM,), jnp.float32),
        mesh=mesh,
        scratch_types=[
            pltpu.VMEM((2, DIM, TILE_C), jnp.float32),
            pltpu.VMEM((2, BLK), jnp.float32),
        ] + [pltpu.SemaphoreType.DMA] * 4,
        compiler_params=pltpu.CompilerParams(
            use_tc_tiling_on_sc=True, needs_layout_passes=False),
    )
    def k1(embt_hbm, tail_hbm, out_hbm, src_v, dst_v, rs0, rs1, ws0, ws1):
        wid = lax.axis_index("s") * NC + lax.axis_index("c")
        rsem = (rs0, rs1)
        wsem = (ws0, ws1)

        @pl.when(wid == 0)
        def _():
            pltpu.sync_copy(tail_hbm, dst_v.at[0, pl.ds(0, TAIL * DIM)])
            pltpu.sync_copy(dst_v.at[0, pl.ds(0, TAIL * DIM)],
                            out_hbm.at[pl.ds(NTC * TILE_C * DIM, TAIL * DIM)])

        def ctile(k):
            return wid + k * NW

        def read(k, b):
            pltpu.async_copy(
                embt_hbm.at[:, pl.ds(ctile(k) * TILE_C, TILE_C)],
                src_v.at[b], rsem[b])

        def rwait(b):
            pltpu.make_async_copy(
                embt_hbm.at[:, pl.ds(0, TILE_C)], src_v.at[b], rsem[b]).wait()

        def write(k, b):
            pltpu.async_copy(
                dst_v.at[b], out_hbm.at[pl.ds(ctile(k) * BLK, BLK)], wsem[b])

        def wwait(b):
            pltpu.make_async_copy(
                out_hbm.at[pl.ds(0, BLK)], dst_v.at[b], wsem[b]).wait()

        def transpose(b):
            def tbody(i, carry):
                cols = jnp.full((L,), i, jnp.int32)
                for q in range(DIM // L):
                    rows = lax.iota(jnp.int32, L) + (q * L)
                    vals = plsc.load_gather(src_v.at[b], [rows, cols])
                    dst_v[b, pl.ds(i * DIM + q * L, L)] = vals
                return carry

            lax.fori_loop(0, TILE_C, tbody, 0, unroll=4)

        @pl.when(ctile(0) < NTC)
        def _():
            read(0, 0)

        def group(g, carry):
            for b in range(2):
                k = g * 2 + b

                @pl.when(ctile(k + 1) < NTC)
                def _():
                    read(k + 1, 1 - b)

                @pl.when(ctile(k) < NTC)
                def _():
                    rwait(b)

                    @pl.when(k >= 2)
                    def _():
                        wwait(b)

                    transpose(b)
                    write(k, b)
            return carry

        lax.fori_loop(0, (K1_ITERS + 2) // 2, group, 0, unroll=False)

        wwait(0)
        wwait(1)

    return k1


def _make_k2():
    mesh = plsc.VectorSubcoreMesh(core_axis_name="c", subcore_axis_name="s")

    @functools.partial(
        pl.kernel,
        out_type=jax.ShapeDtypeStruct((N_TOK, SEQ, DIM), jnp.float32),
        mesh=mesh,
        scratch_types=[
            pltpu.VMEM((ROWS_PER_W, SEQ), jnp.int32),
            pltpu.VMEM((NBUF, SEQ, DIM), jnp.float32),
        ] + [pltpu.SemaphoreType.DMA] * NBUF,
        compiler_params=pltpu.CompilerParams(use_tc_tiling_on_sc=False),
    )
    def k2(tok_hbm, emb_hbm, out_hbm, idx_v, rows_v, *sems):
        wid = lax.axis_index("s") * NC + lax.axis_index("c")
        base = wid * ROWS_PER_W
        pltpu.sync_copy(tok_hbm.at[pl.ds(base, ROWS_PER_W)], idx_v)

        def gather(r, b):
            pltpu.async_copy(emb_hbm.at[idx_v.at[r]], rows_v.at[b], sems[b])

        def drain(b):
            pltpu.make_async_copy(
                emb_hbm.at[pl.ds(0, SEQ)], rows_v.at[b], sems[b]).wait()

        def put(r, b):
            pltpu.sync_copy(rows_v.at[b], out_hbm.at[base + r])

        for c in range(PF):
            gather(c, c)

        def group(g0, carry):
            g = g0 * NBUF
            for b in range(NBUF):
                r = g + b
                drain(b)

                @pl.when(r + PF < ROWS_PER_W)
                def _():
                    gather(r + PF, (b + PF) % NBUF)

                put(r, b)
            return carry

        lax.fori_loop(0, ROWS_PER_W // NBUF, group, 0, unroll=False)

    return k2


_k1 = _make_k1()
_k2 = _make_k2()


def kernel(token_ids, embeddings):
    emb_t = embeddings.T
    tail = embeddings[NTC * TILE_C:].reshape(-1)
    flat = _k1(emb_t, tail)
    table = flat.reshape(NUM_EMB, DIM)
    return _k2(token_ids.astype(jnp.int32), table)


# speedup vs baseline: 2.3978x; 2.3978x over previous
import functools

import jax
import jax.numpy as jnp
from jax import lax
from jax.experimental import pallas as pl
from jax.experimental.pallas import tpu as pltpu
from jax.experimental.pallas import tpu_sc as plsc

NUM_EMB = 1000000
DIM = 64
PDIM = 128
N_TOK = 4096
SEQ = 50
NC = 2
NS = 16
NW = NC * NS
ROWS_PER_W = N_TOK // NW
NBUF = 8
PF = NBUF - 1


def _make_kernel():
    mesh = plsc.VectorSubcoreMesh(core_axis_name="c", subcore_axis_name="s")

    @functools.partial(
        pl.kernel,
        out_type=jax.ShapeDtypeStruct((N_TOK, SEQ, DIM), jnp.float32),
        mesh=mesh,
        scratch_types=[
            pltpu.VMEM((ROWS_PER_W, SEQ), jnp.int32),
            pltpu.VMEM((NBUF, SEQ, PDIM), jnp.float32),
        ] + [pltpu.SemaphoreType.DMA] * NBUF,
        compiler_params=pltpu.CompilerParams(use_tc_tiling_on_sc=False),
    )
    def k2(tok_hbm, emb_hbm, out_hbm, idx_v, rows_v, *sems):
        wid = lax.axis_index("s") * NC + lax.axis_index("c")
        base = wid * ROWS_PER_W
        pltpu.sync_copy(tok_hbm.at[pl.ds(base, ROWS_PER_W)], idx_v)

        def gather(r, b):
            pltpu.async_copy(emb_hbm.at[idx_v.at[r]], rows_v.at[b], sems[b])

        def drain(b):
            pltpu.make_async_copy(
                emb_hbm.at[pl.ds(0, SEQ)], rows_v.at[b], sems[b]).wait()

        def put(r, b):
            pltpu.sync_copy(rows_v.at[b, :, pl.ds(0, DIM)],
                            out_hbm.at[base + r])

        for c in range(PF):
            gather(c, c)

        def group(g0, carry):
            g = g0 * NBUF
            for b in range(NBUF):
                r = g + b
                drain(b)

                @pl.when(r + PF < ROWS_PER_W)
                def _():
                    gather(r + PF, (b + PF) % NBUF)

                put(r, b)
            return carry

        lax.fori_loop(0, ROWS_PER_W // NBUF, group, 0, unroll=False)

    return k2


_k2 = _make_kernel()


def kernel(token_ids, embeddings):
    emb128 = jnp.pad(embeddings, ((0, 0), (0, PDIM - DIM)))
    return _k2(token_ids.astype(jnp.int32), emb128)
